# baseline (speedup 1.0000x reference)
import functools

import jax
import jax.numpy as jnp
from jax import lax
from jax.experimental import pallas as pl
from jax.experimental.pallas import tpu as pltpu
from jax.experimental.pallas import tpu_sc as plsc

_N_ROWS = 64
_D = 128


def _make_gather():
    mesh = plsc.ScalarSubcoreMesh(axis_name="c", num_cores=2)
    half = _N_ROWS // 2

    @functools.partial(
        pl.kernel,
        mesh=mesh,
        out_type=jax.ShapeDtypeStruct((_N_ROWS, _D), jnp.float32),
        scratch_types=[
            pltpu.SMEM((_N_ROWS,), jnp.int32),
            pltpu.SemaphoreType.DMA,
        ],
    )
    def gather_kernel(table_hbm, idx_hbm, out_hbm, idx_s, sem):
        base = lax.axis_index("c") * half
        pltpu.sync_copy(idx_hbm, idx_s)
        for i in range(half):
            pltpu.make_async_copy(
                table_hbm.at[pl.ds(idx_s[base + i], 1)],
                out_hbm.at[pl.ds(base + i, 1)],
                sem).start()
        pltpu.make_async_copy(
            table_hbm.at[pl.ds(0, half)], out_hbm.at[pl.ds(base, half)],
            sem).wait()

    return gather_kernel


_gather = _make_gather()


def kernel(table, idx):
    return _gather(table, idx.astype(jnp.int32))


# speedup vs baseline: 1.0480x; 1.0480x over previous
import functools

import jax
import jax.numpy as jnp
from jax import lax
from jax.experimental import pallas as pl
from jax.experimental.pallas import tpu as pltpu
from jax.experimental.pallas import tpu_sc as plsc

_N_ROWS = 64
_D = 128


def _make_gather():
    mesh = plsc.ScalarSubcoreMesh(axis_name="c", num_cores=1)

    @functools.partial(
        pl.kernel,
        mesh=mesh,
        out_type=jax.ShapeDtypeStruct((_N_ROWS, _D), jnp.float32),
        scratch_types=[
            pltpu.SMEM((_N_ROWS,), jnp.int32),
            pltpu.SemaphoreType.DMA,
        ],
    )
    def gather_kernel(table_hbm, idx_hbm, out_hbm, idx_s, sem):
        pltpu.sync_copy(idx_hbm, idx_s)

        def body(i, carry):
            pltpu.make_async_copy(
                table_hbm.at[pl.ds(idx_s[i], 1)], out_hbm.at[pl.ds(i, 1)],
                sem).start()
            return carry

        lax.fori_loop(0, _N_ROWS, body, 0)
        pltpu.make_async_copy(
            table_hbm.at[pl.ds(0, _N_ROWS)], out_hbm, sem).wait()

    return gather_kernel


_gather = _make_gather()


def kernel(table, idx):
    return _gather(table, idx.astype(jnp.int32))
